# Initial kernel scaffold; baseline (speedup 1.0000x reference)
import functools

import jax
import jax.numpy as jnp
from jax import lax
from jax.experimental import pallas as pl
from jax.experimental.pallas import tpu as pltpu
from jax.experimental.pallas import tpu_sc as plsc

D = 64
CTX = 20
NC, NS, L = 2, 16, 16
NW = NC * NS
CB = 16
NSPLIT = 4


def _sc_scores(ctx_idx, ctr_idx, table):
    TB = ctr_idx.shape[0]
    per_w = TB // NW
    n_chunks = per_w // CB
    nrow = CB * CTX
    gsz = nrow // NSPLIT

    mesh = plsc.VectorSubcoreMesh(core_axis_name="c", subcore_axis_name="s")

    @functools.partial(
        pl.kernel,
        out_type=jax.ShapeDtypeStruct((TB,), jnp.float32),
        mesh=mesh,
        scratch_types=[
            pltpu.VMEM((nrow,), jnp.int32),
            pltpu.VMEM((CB,), jnp.int32),
            pltpu.VMEM((nrow, D), jnp.float32),
            pltpu.VMEM((CB, D), jnp.float32),
            pltpu.VMEM((CB,), jnp.float32),
            pltpu.SemaphoreType.DMA,
        ],
    )
    def k(ctx_hbm, ctr_hbm, tab_hbm, out_hbm, cidx_v, vidx_v, crows_v, vrows_v, sc_v, sem):
        wid = lax.axis_index("s") * NC + lax.axis_index("c")
        base = wid * per_w

        def chunk(g, carry):
            eb = base + g * CB
            pltpu.sync_copy(ctx_hbm.at[pl.ds(eb * CTX, nrow)], cidx_v)
            pltpu.sync_copy(ctr_hbm.at[pl.ds(eb, CB)], vidx_v)
            copies = [
                pltpu.async_copy(
                    tab_hbm.at[cidx_v.at[pl.ds(j * gsz, gsz)]],
                    crows_v.at[pl.ds(j * gsz, gsz)],
                    sem,
                )
                for j in range(NSPLIT)
            ]
            copies.append(pltpu.async_copy(tab_hbm.at[vidx_v], vrows_v, sem))
            for cp in copies:
                cp.wait()

            lane = lax.iota(jnp.int32, L)
            sv = jnp.zeros((L,), jnp.float32)
            for i in range(CB):
                acc = [crows_v[i * CTX, pl.ds(kk * L, L)] for kk in range(D // L)]
                for c in range(1, CTX):
                    for kk in range(D // L):
                        acc[kk] = acc[kk] + crows_v[i * CTX + c, pl.ds(kk * L, L)]
                p = acc[0] * vrows_v[i, pl.ds(0, L)]
                for kk in range(1, D // L):
                    p = p + acc[kk] * vrows_v[i, pl.ds(kk * L, L)]
                s = jnp.sum(p)
                sv = jnp.where(lane == i, s, sv)
            sc_v[...] = sv
            pltpu.sync_copy(sc_v, out_hbm.at[pl.ds(eb, CB)])
            return carry

        lax.fori_loop(0, n_chunks, chunk, 0)

    return k(ctx_idx, ctr_idx, table)


def _tc_loss(scores):
    n = scores.shape[0]
    x2 = scores.reshape(n // 128, 128)
    half = n // 256

    def body(x_ref, o_ref):
        x = x_ref[...]
        row = lax.broadcasted_iota(jnp.int32, x.shape, 0)
        y = jnp.where(row < half, x, -x)
        ls = jnp.minimum(y, 0.0) - jnp.log1p(jnp.exp(-jnp.abs(y)))
        o_ref[0, 0] = -jnp.sum(ls)

    out = pl.pallas_call(
        body,
        out_shape=jax.ShapeDtypeStruct((1, 1), jnp.float32),
        out_specs=pl.BlockSpec(memory_space=pltpu.SMEM),
    )(x2)
    return out.reshape(())


def kernel(pos_u, pos_v, neg_u, neg_v, u_table, v_table):
    ctx_idx = jnp.concatenate([pos_u.reshape(-1), neg_u.reshape(-1)])
    ctr_idx = jnp.concatenate([pos_v, neg_v])
    scores = _sc_scores(ctx_idx, ctr_idx, u_table)
    return _tc_loss(scores)


# speedup vs baseline: 1.1381x; 1.1381x over previous
import functools

import jax
import jax.numpy as jnp
from jax import lax
from jax.experimental import pallas as pl
from jax.experimental.pallas import tpu as pltpu
from jax.experimental.pallas import tpu_sc as plsc

D = 64
CTX = 20
NC, NS, L = 2, 16, 16
NW = NC * NS
CB = 16
NSPLIT = 4


def _sc_scores(ctx_idx, ctr_idx, table):
    TB = ctr_idx.shape[0]
    per_w = TB // NW
    n_chunks = per_w // CB
    nrow = CB * CTX
    gsz = nrow // NSPLIT

    mesh = plsc.VectorSubcoreMesh(core_axis_name="c", subcore_axis_name="s")

    @functools.partial(
        pl.kernel,
        out_type=jax.ShapeDtypeStruct((TB,), jnp.float32),
        mesh=mesh,
        scratch_types=[
            pltpu.VMEM((nrow,), jnp.int32),
            pltpu.VMEM((CB,), jnp.int32),
            pltpu.VMEM((nrow, D), jnp.float32),
            pltpu.VMEM((CB, D), jnp.float32),
            pltpu.VMEM((L * CB,), jnp.float32),
            pltpu.VMEM((CB,), jnp.float32),
            pltpu.SemaphoreType.DMA,
        ],
        compiler_params=pltpu.CompilerParams(
            needs_layout_passes=False, use_tc_tiling_on_sc=False
        ),
    )
    def k(ctx_hbm, ctr_hbm, tab_hbm, out_hbm, cidx_v, vidx_v, crows_v, vrows_v, pbuf_v, sc_v, sem):
        wid = lax.axis_index("s") * NC + lax.axis_index("c")
        base = wid * per_w

        def chunk(g, carry):
            eb = base + g * CB
            pltpu.sync_copy(ctx_hbm.at[pl.ds(eb * CTX, nrow)], cidx_v)
            pltpu.sync_copy(ctr_hbm.at[pl.ds(eb, CB)], vidx_v)
            copies = [
                pltpu.async_copy(
                    tab_hbm.at[cidx_v.at[pl.ds(j * gsz, gsz)]],
                    crows_v.at[pl.ds(j * gsz, gsz)],
                    sem,
                )
                for j in range(NSPLIT)
            ]
            copies.append(pltpu.async_copy(tab_hbm.at[vidx_v], vrows_v, sem))
            for cp in copies:
                cp.wait()

            lane = lax.iota(jnp.int32, L)
            for i in range(CB):
                acc = [crows_v[i * CTX, pl.ds(kk * L, L)] for kk in range(D // L)]
                for c in range(1, CTX):
                    for kk in range(D // L):
                        acc[kk] = acc[kk] + crows_v[i * CTX + c, pl.ds(kk * L, L)]
                p = acc[0] * vrows_v[i, pl.ds(0, L)]
                for kk in range(1, D // L):
                    p = p + acc[kk] * vrows_v[i, pl.ds(kk * L, L)]
                plsc.store_scatter(pbuf_v, [lane * CB + i], p)
            sv = pbuf_v[pl.ds(0, L)]
            for d in range(1, L):
                sv = sv + pbuf_v[pl.ds(d * CB, L)]
            sc_v[...] = sv
            pltpu.sync_copy(sc_v, out_hbm.at[pl.ds(eb, CB)])
            return carry

        lax.fori_loop(0, n_chunks, chunk, 0)

    return k(ctx_idx, ctr_idx, table)


def _tc_loss(scores):
    n = scores.shape[0]
    x2 = scores.reshape(n // 128, 128)
    half = n // 256

    def body(x_ref, o_ref):
        x = x_ref[...]
        row = lax.broadcasted_iota(jnp.int32, x.shape, 0)
        y = jnp.where(row < half, x, -x)
        ls = jnp.minimum(y, 0.0) - jnp.log1p(jnp.exp(-jnp.abs(y)))
        o_ref[0, 0] = -jnp.sum(ls)

    out = pl.pallas_call(
        body,
        out_shape=jax.ShapeDtypeStruct((1, 1), jnp.float32),
        out_specs=pl.BlockSpec(memory_space=pltpu.SMEM),
    )(x2)
    return out.reshape(())


def kernel(pos_u, pos_v, neg_u, neg_v, u_table, v_table):
    ctx_idx = jnp.concatenate([pos_u.reshape(-1), neg_u.reshape(-1)])
    ctr_idx = jnp.concatenate([pos_v, neg_v])
    scores = _sc_scores(ctx_idx, ctr_idx, u_table)
    return _tc_loss(scores)


# speedup vs baseline: 1.2982x; 1.1407x over previous
import functools

import jax
import jax.numpy as jnp
from jax import lax
from jax.experimental import pallas as pl
from jax.experimental.pallas import tpu as pltpu
from jax.experimental.pallas import tpu_sc as plsc

D = 64
CTX = 20
NC, NS, L = 2, 16, 16
NW = NC * NS
CB = 16
NSPLIT = 4


def _sc_scores(ctx_idx, ctr_idx, table):
    TB = ctr_idx.shape[0]
    per_w = TB // NW
    n_chunks = per_w // CB
    nrow = CB * CTX
    gsz = nrow // NSPLIT

    mesh = plsc.VectorSubcoreMesh(core_axis_name="c", subcore_axis_name="s")

    @functools.partial(
        pl.kernel,
        out_type=jax.ShapeDtypeStruct((TB,), jnp.float32),
        mesh=mesh,
        scratch_types=[
            pltpu.VMEM((per_w * CTX,), jnp.int32),
            pltpu.VMEM((per_w,), jnp.int32),
            pltpu.VMEM((2, nrow, D), jnp.float32),
            pltpu.VMEM((2, CB, D), jnp.float32),
            pltpu.VMEM((L * CB,), jnp.float32),
            pltpu.VMEM((CB,), jnp.float32),
            pltpu.SemaphoreType.DMA,
            pltpu.SemaphoreType.DMA,
        ],
        compiler_params=pltpu.CompilerParams(
            needs_layout_passes=False, use_tc_tiling_on_sc=False
        ),
    )
    def k(ctx_hbm, ctr_hbm, tab_hbm, out_hbm, cidx_v, vidx_v, crows_v, vrows_v, pbuf_v, sc_v, sem0, sem1):
        wid = lax.axis_index("s") * NC + lax.axis_index("c")
        base = wid * per_w
        sems = (sem0, sem1)

        pltpu.sync_copy(ctx_hbm.at[pl.ds(base * CTX, per_w * CTX)], cidx_v)
        pltpu.sync_copy(ctr_hbm.at[pl.ds(base, per_w)], vidx_v)

        def fire(g, b):
            for j in range(NSPLIT):
                pltpu.async_copy(
                    tab_hbm.at[cidx_v.at[pl.ds(g * nrow + j * gsz, gsz)]],
                    crows_v.at[b, pl.ds(j * gsz, gsz)],
                    sems[b],
                )
            pltpu.async_copy(
                tab_hbm.at[vidx_v.at[pl.ds(g * CB, CB)]],
                vrows_v.at[b],
                sems[b],
            )

        def drain(g, b):
            for j in range(NSPLIT):
                pltpu.make_async_copy(
                    tab_hbm.at[cidx_v.at[pl.ds(g * nrow + j * gsz, gsz)]],
                    crows_v.at[b, pl.ds(j * gsz, gsz)],
                    sems[b],
                ).wait()
            pltpu.make_async_copy(
                tab_hbm.at[vidx_v.at[pl.ds(g * CB, CB)]],
                vrows_v.at[b],
                sems[b],
            ).wait()

        fire(0, 0)
        fire(1, 1)

        lane = lax.iota(jnp.int32, L)

        def body(g2, carry):
            for b in range(2):
                g = g2 * 2 + b
                drain(g, b)
                for i in range(CB):
                    acc = [crows_v[b, i * CTX, pl.ds(kk * L, L)] for kk in range(D // L)]
                    for c in range(1, CTX):
                        for kk in range(D // L):
                            acc[kk] = acc[kk] + crows_v[b, i * CTX + c, pl.ds(kk * L, L)]
                    p = acc[0] * vrows_v[b, i, pl.ds(0, L)]
                    for kk in range(1, D // L):
                        p = p + acc[kk] * vrows_v[b, i, pl.ds(kk * L, L)]
                    plsc.store_scatter(pbuf_v, [lane * CB + i], p)
                sv = pbuf_v[pl.ds(0, L)]
                for d in range(1, L):
                    sv = sv + pbuf_v[pl.ds(d * CB, L)]
                sc_v[...] = sv
                pltpu.sync_copy(sc_v, out_hbm.at[pl.ds(base + g * CB, CB)])

                @pl.when(g + 2 < n_chunks)
                def _():
                    fire(g + 2, b)

            return carry

        lax.fori_loop(0, n_chunks // 2, body, 0)

    return k(ctx_idx, ctr_idx, table)


def _tc_loss(scores):
    n = scores.shape[0]
    x2 = scores.reshape(n // 128, 128)
    half = n // 256

    def body(x_ref, o_ref):
        x = x_ref[...]
        row = lax.broadcasted_iota(jnp.int32, x.shape, 0)
        y = jnp.where(row < half, x, -x)
        ls = jnp.minimum(y, 0.0) - jnp.log1p(jnp.exp(-jnp.abs(y)))
        o_ref[0, 0] = -jnp.sum(ls)

    out = pl.pallas_call(
        body,
        out_shape=jax.ShapeDtypeStruct((1, 1), jnp.float32),
        out_specs=pl.BlockSpec(memory_space=pltpu.SMEM),
    )(x2)
    return out.reshape(())


def kernel(pos_u, pos_v, neg_u, neg_v, u_table, v_table):
    ctx_idx = jnp.concatenate([pos_u.reshape(-1), neg_u.reshape(-1)])
    ctr_idx = jnp.concatenate([pos_v, neg_v])
    scores = _sc_scores(ctx_idx, ctr_idx, u_table)
    return _tc_loss(scores)
